# baseline (speedup 1.0000x reference)
import functools

import jax
import jax.numpy as jnp
from jax import lax
from jax.experimental import pallas as pl
from jax.experimental.pallas import tpu as pltpu
from jax.experimental.pallas import tpu_sc as plsc

NN = 10000
EDGES = 320000
H1 = 64
H2 = 128
OUTC = 32
NCORE = 2
NSUB = 16
NW = NCORE * NSUB
EPT = EDGES // NW
CHUNK = 80
NCHUNK = EPT // CHUNK
NNP = 10240
RPT = NNP // NSUB
RCH = 128
NRCH = RPT // RCH

_MESH = dict(core_axis_name="c", subcore_axis_name="s")


def _make_agg():
    C = 128
    mesh = plsc.VectorSubcoreMesh(**_MESH)

    @functools.partial(
        pl.kernel, mesh=mesh,
        out_type=jax.ShapeDtypeStruct((NCORE, NNP, C), jnp.float32),
        scratch_types=[
            pltpu.VMEM((CHUNK,), jnp.int32),
            pltpu.VMEM((CHUNK,), jnp.int32),
            pltpu.VMEM((CHUNK, C), jnp.float32),
            pltpu.VMEM((RCH, C), jnp.float32),
            pltpu.VMEM_SHARED((NNP, C), jnp.float32),
            pltpu.SemaphoreType.DMA,
        ],
    )
    def agg(gid_hbm, dst_hbm, table_hbm, out_hbm, gidv, dstv, rows, zbuf,
            acc, sem):
        c = lax.axis_index("c")
        s = lax.axis_index("s")
        wid = s * NCORE + c
        zero = jnp.zeros((16,), jnp.float32)

        def zrow(r, carry):
            for j in range(C // 16):
                zbuf[r, pl.ds(j * 16, 16)] = zero
            return carry

        lax.fori_loop(0, RCH, zrow, 0)

        def zcp(k, carry):
            pltpu.sync_copy(zbuf, acc.at[pl.ds(s * RPT + k * RCH, RCH)])
            return carry

        lax.fori_loop(0, NRCH, zcp, 0)
        plsc.subcore_barrier()

        base = wid * EPT

        def chunk(k, carry):
            off = base + k * CHUNK
            pltpu.sync_copy(gid_hbm.at[pl.ds(off, CHUNK)], gidv)
            pltpu.sync_copy(dst_hbm.at[pl.ds(off, CHUNK)], dstv)
            pltpu.async_copy(table_hbm.at[gidv], rows, sem).wait()
            pltpu.sync_copy(rows, acc.at[dstv], add=True)
            return carry

        lax.fori_loop(0, NCHUNK, chunk, 0)
        plsc.subcore_barrier()

        def wout(k, carry):
            r0 = s * RPT + k * RCH
            pltpu.sync_copy(acc.at[pl.ds(r0, RCH)], zbuf)
            pltpu.sync_copy(zbuf, out_hbm.at[c, pl.ds(r0, RCH)])
            return carry

        lax.fori_loop(0, NRCH, wout, 0)

    return agg


def _make_deg():
    mesh = plsc.VectorSubcoreMesh(**_MESH)

    @functools.partial(
        pl.kernel, mesh=mesh,
        out_type=jax.ShapeDtypeStruct((NCORE, NNP, 128), jnp.float32),
        scratch_types=[
            pltpu.VMEM((CHUNK,), jnp.int32),
            pltpu.VMEM((CHUNK, 128), jnp.float32),
            pltpu.VMEM((RCH, 128), jnp.float32),
            pltpu.VMEM_SHARED((NNP, 128), jnp.float32),
        ],
    )
    def deg(dst_hbm, out_hbm, dstv, ones_rows, zbuf, acc):
        c = lax.axis_index("c")
        s = lax.axis_index("s")
        wid = s * NCORE + c
        zero = jnp.zeros((16,), jnp.float32)
        one = jnp.ones((16,), jnp.float32)

        def frow(r, carry):
            for j in range(128 // 16):
                ones_rows[r, pl.ds(j * 16, 16)] = one
            return carry

        lax.fori_loop(0, CHUNK, frow, 0)

        def frow2(r, carry):
            for j in range(128 // 16):
                zbuf[r, pl.ds(j * 16, 16)] = zero
            return carry

        lax.fori_loop(0, RCH, frow2, 0)

        def zcp(k, carry):
            pltpu.sync_copy(zbuf, acc.at[pl.ds(s * RPT + k * RCH, RCH)])
            return carry

        lax.fori_loop(0, NRCH, zcp, 0)
        plsc.subcore_barrier()

        base = wid * EPT

        def chunk(k, carry):
            pltpu.sync_copy(dst_hbm.at[pl.ds(base + k * CHUNK, CHUNK)], dstv)
            pltpu.sync_copy(ones_rows, acc.at[dstv], add=True)
            return carry

        lax.fori_loop(0, NCHUNK, chunk, 0)
        plsc.subcore_barrier()

        def wout(k, carry):
            r0 = s * RPT + k * RCH
            pltpu.sync_copy(acc.at[pl.ds(r0, RCH)], zbuf)
            pltpu.sync_copy(zbuf, out_hbm.at[c, pl.ds(r0, RCH)])
            return carry

        lax.fori_loop(0, NRCH, wout, 0)

    return deg


def _dinv_body(da_ref, dinv_ref):
    deg = da_ref[0, :NN, 0:1] + da_ref[1, :NN, 0:1]
    y = lax.rsqrt(deg)
    dinv_ref[...] = jnp.where(deg > 0.0, y, 0.0)


def _init_body(xc_ref, dinv_ref, r_ref, b_ref, z_ref, o_ref):
    x = xc_ref[...]
    z_ref[...] = dinv_ref[...] * x
    o_ref[...] = (jnp.dot(x, r_ref[...], preferred_element_type=jnp.float32)
                  + b_ref[...])


def _sum_body(o_ref, p0_ref, p1_ref, p2_ref, dinv_ref, w_ref, u_ref):
    cin = w_ref.shape[1]
    dinv = dinv_ref[...]
    acc = o_ref[...]
    for i, p_ref in enumerate((p0_ref, p1_ref, p2_ref)):
        h = dinv * (p_ref[0, :, :cin] + p_ref[1, :, :cin])
        acc = acc + jnp.dot(h, w_ref[i], preferred_element_type=jnp.float32)
    u_ref[...] = acc


def _bnr_body(u_ref, dinv_ref, g_ref, be_ref, r_ref, b_ref, z_ref, o_ref):
    h = u_ref[...]
    mu = jnp.mean(h, axis=0, keepdims=True)
    var = jnp.mean(jnp.square(h - mu), axis=0, keepdims=True)
    a = jnp.maximum(
        g_ref[...] * (h - mu) * lax.rsqrt(var + 1e-5) + be_ref[...], 0.0)
    z = dinv_ref[...] * a
    cw = z.shape[1]
    z_ref[:, :cw] = z
    if cw < 128:
        z_ref[:, cw:] = jnp.zeros((z.shape[0], 128 - cw), jnp.float32)
    o_ref[...] = (jnp.dot(a, r_ref[...], preferred_element_type=jnp.float32)
                  + b_ref[...])


def _dinv_k(da):
    return pl.pallas_call(
        _dinv_body,
        out_shape=jax.ShapeDtypeStruct((NN, 1), jnp.float32),
    )(da)


def _init(xc, dinv, r, b):
    return pl.pallas_call(
        _init_body,
        out_shape=(jax.ShapeDtypeStruct((NN, 128), jnp.float32),
                   jax.ShapeDtypeStruct((NN, r.shape[1]), jnp.float32)),
    )(xc, dinv, r, b)


BROW = 2000


def _sum(o, p0, p1, p2, dinv, w):
    cin, co = w.shape[1], w.shape[2]
    pspec = pl.BlockSpec((2, BROW, 128), lambda i: (0, i, 0))
    return pl.pallas_call(
        _sum_body,
        grid=(NN // BROW,),
        in_specs=[
            pl.BlockSpec((BROW, co), lambda i: (i, 0)),
            pspec, pspec, pspec,
            pl.BlockSpec((BROW, 1), lambda i: (i, 0)),
            pl.BlockSpec((3, cin, co), lambda i: (0, 0, 0)),
        ],
        out_specs=pl.BlockSpec((BROW, co), lambda i: (i, 0)),
        out_shape=jax.ShapeDtypeStruct((NN, co), jnp.float32),
    )(o, p0[:, :NN], p1[:, :NN], p2[:, :NN], dinv, w)


def _bnr(u, dinv, g, be, r, b):
    return pl.pallas_call(
        _bnr_body,
        out_shape=(jax.ShapeDtypeStruct((NN, 128), jnp.float32),
                   jax.ShapeDtypeStruct((NN, r.shape[1]), jnp.float32)),
    )(u, dinv, g, be, r, b)


_deg_k = _make_deg()
_agg = _make_agg()


def _agg3(gids, dsts, z):
    return [_agg(gids[i], dsts[i], z) for i in range(3)]


def kernel(x, adj_t, edge_types, emb, w1, r1, b1, w2, r2, b2, wi, ri, bi,
           w3, r3, b3, g1, be1, g2, be2):
    xc = jnp.concatenate([x, emb], axis=0)
    src = adj_t[0]
    dst = adj_t[1]
    gids = [jnp.where(edge_types == i, src, 0) for i in range(3)]
    dsts = [jnp.where(edge_types == i, dst, NN) for i in range(3)]

    dinv = _dinv_k(_deg_k(dst))

    b2r, bir, b3r = (v.reshape(1, -1) for v in (b2, bi, b3))
    g1r, be1r, g2r, be2r = (v.reshape(1, -1) for v in (g1, be1, g2, be2))
    b1r = b1.reshape(1, -1)

    z, o = _init(xc, dinv, r1, b1r)
    layers = [(w1, g1r, be1r, r2, b2r),
              (w2, g2r, be2r, ri, bir),
              (wi, g2r, be2r, ri, bir),
              (wi, g2r, be2r, ri, bir),
              (wi, g2r, be2r, ri, bir),
              (wi, g2r, be2r, ri, bir),
              (wi, g2r, be2r, r3, b3r)]
    for w, g, be, rn, bn_ in layers:
        p0, p1, p2 = _agg3(gids, dsts, z)
        u = _sum(o, p0, p1, p2, dinv, w)
        z, o = _bnr(u, dinv, g, be, rn, bn_)
    p0, p1, p2 = _agg3(gids, dsts, z)
    return _sum(o, p0, p1, p2, dinv, w3)


# speedup vs baseline: 1.0000x; 1.0000x over previous
import functools

import jax
import jax.numpy as jnp
from jax import lax
from jax.experimental import pallas as pl
from jax.experimental.pallas import tpu as pltpu
from jax.experimental.pallas import tpu_sc as plsc

NN = 10000
EDGES = 320000
H1 = 64
H2 = 128
OUTC = 32
NCORE = 2
NSUB = 16
NW = NCORE * NSUB
EPT = EDGES // NW
CHUNK = 40
NCHUNK = EPT // CHUNK
NNP = 10240
RPT = NNP // NSUB
RCH = 64
NRCH = RPT // RCH

_MESH = dict(core_axis_name="c", subcore_axis_name="s")


def _make_agg():
    C = 128
    mesh = plsc.VectorSubcoreMesh(**_MESH)

    @functools.partial(
        pl.kernel, mesh=mesh,
        out_type=jax.ShapeDtypeStruct((NCORE, NNP, C), jnp.float32),
        scratch_types=[
            pltpu.VMEM((CHUNK,), jnp.int32),
            pltpu.VMEM((CHUNK,), jnp.int32),
            pltpu.VMEM((CHUNK,), jnp.int32),
            pltpu.VMEM((CHUNK,), jnp.int32),
            pltpu.VMEM((CHUNK, C), jnp.float32),
            pltpu.VMEM((CHUNK, C), jnp.float32),
            pltpu.VMEM((RCH, C), jnp.float32),
            pltpu.VMEM_SHARED((NNP, C), jnp.float32),
            pltpu.SemaphoreType.DMA,
            pltpu.SemaphoreType.DMA,
            pltpu.SemaphoreType.DMA,
            pltpu.SemaphoreType.DMA,
        ],
    )
    def agg(gid_hbm, dst_hbm, table_hbm, out_hbm, g0, d0, g1, d1,
            rows0, rows1, zbuf, acc, si0, si1, sg0, sg1):
        c = lax.axis_index("c")
        s = lax.axis_index("s")
        wid = s * NCORE + c
        base = wid * EPT
        zero = jnp.zeros((16,), jnp.float32)

        def zrow(r, carry):
            for j in range(C // 16):
                zbuf[r, pl.ds(j * 16, 16)] = zero
            return carry

        lax.fori_loop(0, RCH, zrow, 0)

        def zcp(k, carry):
            pltpu.sync_copy(zbuf, acc.at[pl.ds(s * RPT + k * RCH, RCH)])
            return carry

        lax.fori_loop(0, NRCH, zcp, 0)
        plsc.subcore_barrier()

        def idx_start(k, gb, db, sem):
            off = base + k * CHUNK
            pltpu.async_copy(gid_hbm.at[pl.ds(off, CHUNK)], gb, sem)
            pltpu.async_copy(dst_hbm.at[pl.ds(off, CHUNK)], db, sem)

        def idx_wait(k, gb, db, sem):
            off = base + k * CHUNK
            pltpu.make_async_copy(gid_hbm.at[pl.ds(off, CHUNK)], gb,
                                  sem).wait()
            pltpu.make_async_copy(dst_hbm.at[pl.ds(off, CHUNK)], db,
                                  sem).wait()

        idx_start(0, g0, d0, si0)
        idx_wait(0, g0, d0, si0)
        pltpu.async_copy(table_hbm.at[g0], rows0, sg0)
        idx_start(1, g1, d1, si1)

        def body(j, carry):
            k0 = 2 * j
            idx_wait(k0 + 1, g1, d1, si1)
            pltpu.make_async_copy(table_hbm.at[g0], rows0, sg0).wait()
            pltpu.async_copy(table_hbm.at[g1], rows1, sg1)
            pltpu.sync_copy(rows0, acc.at[d0], add=True)
            idx_start(k0 + 2, g0, d0, si0)
            idx_wait(k0 + 2, g0, d0, si0)
            pltpu.make_async_copy(table_hbm.at[g1], rows1, sg1).wait()
            pltpu.async_copy(table_hbm.at[g0], rows0, sg0)
            pltpu.sync_copy(rows1, acc.at[d1], add=True)
            @pl.when(k0 + 3 < NCHUNK)
            def _():
                idx_start(k0 + 3, g1, d1, si1)
            return carry

        lax.fori_loop(0, NCHUNK // 2 - 1, body, 0)
        idx_wait(NCHUNK - 1, g1, d1, si1)
        pltpu.make_async_copy(table_hbm.at[g0], rows0, sg0).wait()
        pltpu.async_copy(table_hbm.at[g1], rows1, sg1)
        pltpu.sync_copy(rows0, acc.at[d0], add=True)
        pltpu.make_async_copy(table_hbm.at[g1], rows1, sg1).wait()
        pltpu.sync_copy(rows1, acc.at[d1], add=True)
        plsc.subcore_barrier()

        def wout(k, carry):
            r0 = s * RPT + k * RCH
            pltpu.sync_copy(acc.at[pl.ds(r0, RCH)], zbuf)
            pltpu.sync_copy(zbuf, out_hbm.at[c, pl.ds(r0, RCH)])
            return carry

        lax.fori_loop(0, NRCH, wout, 0)

    return agg


def _make_deg():
    mesh = plsc.VectorSubcoreMesh(**_MESH)

    @functools.partial(
        pl.kernel, mesh=mesh,
        out_type=jax.ShapeDtypeStruct((NCORE, NNP, 128), jnp.float32),
        scratch_types=[
            pltpu.VMEM((CHUNK,), jnp.int32),
            pltpu.VMEM((CHUNK,), jnp.int32),
            pltpu.VMEM((CHUNK, 128), jnp.float32),
            pltpu.VMEM((RCH, 128), jnp.float32),
            pltpu.VMEM_SHARED((NNP, 128), jnp.float32),
            pltpu.SemaphoreType.DMA,
            pltpu.SemaphoreType.DMA,
        ],
    )
    def deg(dst_hbm, out_hbm, d0, d1, ones_rows, zbuf, acc, si0, si1):
        c = lax.axis_index("c")
        s = lax.axis_index("s")
        wid = s * NCORE + c
        base = wid * EPT
        zero = jnp.zeros((16,), jnp.float32)
        one = jnp.ones((16,), jnp.float32)

        def frow(r, carry):
            for j in range(128 // 16):
                ones_rows[r, pl.ds(j * 16, 16)] = one
            return carry

        lax.fori_loop(0, CHUNK, frow, 0)

        def frow2(r, carry):
            for j in range(128 // 16):
                zbuf[r, pl.ds(j * 16, 16)] = zero
            return carry

        lax.fori_loop(0, RCH, frow2, 0)

        def zcp(k, carry):
            pltpu.sync_copy(zbuf, acc.at[pl.ds(s * RPT + k * RCH, RCH)])
            return carry

        lax.fori_loop(0, NRCH, zcp, 0)
        plsc.subcore_barrier()

        def istart(k, db, sem):
            pltpu.async_copy(dst_hbm.at[pl.ds(base + k * CHUNK, CHUNK)],
                             db, sem)

        def iwait(k, db, sem):
            pltpu.make_async_copy(
                dst_hbm.at[pl.ds(base + k * CHUNK, CHUNK)], db, sem).wait()

        istart(0, d0, si0)

        def chunk(j, carry):
            k0 = 2 * j
            iwait(k0, d0, si0)
            istart(k0 + 1, d1, si1)
            pltpu.sync_copy(ones_rows, acc.at[d0], add=True)
            iwait(k0 + 1, d1, si1)
            @pl.when(k0 + 2 < NCHUNK)
            def _():
                istart(k0 + 2, d0, si0)
            pltpu.sync_copy(ones_rows, acc.at[d1], add=True)
            return carry

        lax.fori_loop(0, NCHUNK // 2, chunk, 0)
        plsc.subcore_barrier()

        def wout(k, carry):
            r0 = s * RPT + k * RCH
            pltpu.sync_copy(acc.at[pl.ds(r0, RCH)], zbuf)
            pltpu.sync_copy(zbuf, out_hbm.at[c, pl.ds(r0, RCH)])
            return carry

        lax.fori_loop(0, NRCH, wout, 0)

    return deg


def _dinv_body(da_ref, dinv_ref):
    deg = da_ref[0, :NN, 0:1] + da_ref[1, :NN, 0:1]
    y = lax.rsqrt(deg)
    dinv_ref[...] = jnp.where(deg > 0.0, y, 0.0)


def _init_body(xc_ref, dinv_ref, r_ref, b_ref, z_ref, o_ref):
    x = xc_ref[...]
    z_ref[...] = dinv_ref[...] * x
    o_ref[...] = (jnp.dot(x, r_ref[...], preferred_element_type=jnp.float32)
                  + b_ref[...])


def _sum_body(o_ref, p0_ref, p1_ref, p2_ref, dinv_ref, w_ref, u_ref):
    cin = w_ref.shape[1]
    dinv = dinv_ref[...]
    acc = o_ref[...]
    for i, p_ref in enumerate((p0_ref, p1_ref, p2_ref)):
        h = dinv * (p_ref[0, :, :cin] + p_ref[1, :, :cin])
        acc = acc + jnp.dot(h, w_ref[i], preferred_element_type=jnp.float32)
    u_ref[...] = acc


def _bnr_body(u_ref, dinv_ref, g_ref, be_ref, r_ref, b_ref, z_ref, o_ref):
    h = u_ref[...]
    mu = jnp.mean(h, axis=0, keepdims=True)
    var = jnp.mean(jnp.square(h - mu), axis=0, keepdims=True)
    a = jnp.maximum(
        g_ref[...] * (h - mu) * lax.rsqrt(var + 1e-5) + be_ref[...], 0.0)
    z = dinv_ref[...] * a
    cw = z.shape[1]
    z_ref[:, :cw] = z
    if cw < 128:
        z_ref[:, cw:] = jnp.zeros((z.shape[0], 128 - cw), jnp.float32)
    o_ref[...] = (jnp.dot(a, r_ref[...], preferred_element_type=jnp.float32)
                  + b_ref[...])


def _dinv_k(da):
    return pl.pallas_call(
        _dinv_body,
        out_shape=jax.ShapeDtypeStruct((NN, 1), jnp.float32),
    )(da)


def _init(xc, dinv, r, b):
    return pl.pallas_call(
        _init_body,
        out_shape=(jax.ShapeDtypeStruct((NN, 128), jnp.float32),
                   jax.ShapeDtypeStruct((NN, r.shape[1]), jnp.float32)),
    )(xc, dinv, r, b)


BROW = 2000


def _sum(o, p0, p1, p2, dinv, w):
    cin, co = w.shape[1], w.shape[2]
    pspec = pl.BlockSpec((2, BROW, 128), lambda i: (0, i, 0))
    return pl.pallas_call(
        _sum_body,
        grid=(NN // BROW,),
        in_specs=[
            pl.BlockSpec((BROW, co), lambda i: (i, 0)),
            pspec, pspec, pspec,
            pl.BlockSpec((BROW, 1), lambda i: (i, 0)),
            pl.BlockSpec((3, cin, co), lambda i: (0, 0, 0)),
        ],
        out_specs=pl.BlockSpec((BROW, co), lambda i: (i, 0)),
        out_shape=jax.ShapeDtypeStruct((NN, co), jnp.float32),
    )(o, p0[:, :NN], p1[:, :NN], p2[:, :NN], dinv, w)


def _bnr(u, dinv, g, be, r, b):
    return pl.pallas_call(
        _bnr_body,
        out_shape=(jax.ShapeDtypeStruct((NN, 128), jnp.float32),
                   jax.ShapeDtypeStruct((NN, r.shape[1]), jnp.float32)),
    )(u, dinv, g, be, r, b)


_deg_k = _make_deg()
_agg = _make_agg()


def _agg3(gids, dsts, z):
    return [_agg(gids[i], dsts[i], z) for i in range(3)]


def kernel(x, adj_t, edge_types, emb, w1, r1, b1, w2, r2, b2, wi, ri, bi,
           w3, r3, b3, g1, be1, g2, be2):
    xc = jnp.concatenate([x, emb], axis=0)
    src = adj_t[0]
    dst = adj_t[1]
    gids = [jnp.where(edge_types == i, src, 0) for i in range(3)]
    dsts = [jnp.where(edge_types == i, dst, NN) for i in range(3)]

    dinv = _dinv_k(_deg_k(dst))

    b2r, bir, b3r = (v.reshape(1, -1) for v in (b2, bi, b3))
    g1r, be1r, g2r, be2r = (v.reshape(1, -1) for v in (g1, be1, g2, be2))
    b1r = b1.reshape(1, -1)

    z, o = _init(xc, dinv, r1, b1r)
    layers = [(w1, g1r, be1r, r2, b2r),
              (w2, g2r, be2r, ri, bir),
              (wi, g2r, be2r, ri, bir),
              (wi, g2r, be2r, ri, bir),
              (wi, g2r, be2r, ri, bir),
              (wi, g2r, be2r, ri, bir),
              (wi, g2r, be2r, r3, b3r)]
    for w, g, be, rn, bn_ in layers:
        p0, p1, p2 = _agg3(gids, dsts, z)
        u = _sum(o, p0, p1, p2, dinv, w)
        z, o = _bnr(u, dinv, g, be, rn, bn_)
    p0, p1, p2 = _agg3(gids, dsts, z)
    return _sum(o, p0, p1, p2, dinv, w3)
